# baseline (speedup 1.0000x reference)
import jax
import jax.numpy as jnp
from jax import lax
from jax.experimental import pallas as pl
from jax.experimental.pallas import tpu as pltpu
from jax.experimental.pallas import tpu_sc as plsc

N_STATES = 1000000
N_ACTIONS = 64
BATCH = 16384

NC = 2
NS = 16
NW = NC * NS
NBLK = (N_STATES + 127) // 128
TAIL0 = (N_STATES // 128) * 128
CBLK = 2
CST = CBLK * 128
MAXBLK = (N_STATES - CST) // 128
NBANK = 4
NCH = 124
NQ = N_ACTIONS // 16
WLCAP = BATCH + 16


def _gather_body(x_hbm, table_hbm, tail_hbm, out_hbm,
                 xs_v, wls_v, wlb_v, tail_v, bufs_v, hs_v, hb_v, rows_v,
                 sems, osems):
    wid = lax.axis_index("s") * NC + lax.axis_index("c")
    lo_blk = (NBLK * wid) >> 5
    lo_s = lo_blk * 128
    hi_s = ((NBLK * (wid + 1)) >> 5) * 128
    pltpu.sync_copy(x_hbm, xs_v)
    pltpu.sync_copy(tail_hbm, tail_v)

    lanes = lax.iota(jnp.int32, 16)
    phi = [(lanes + 16 * q) >> 3 for q in range(NQ)]
    plo = [(lanes + 16 * q) & 7 for q in range(NQ)]

    def bgroup(g, n):
        sv = xs_v[pl.ds(g * 16, 16)]
        bv = lanes + g * 16
        m = jnp.logical_and(sv >= lo_s, sv < hi_s)
        plsc.store_compressed(wls_v.at[pl.ds(n, 16)], sv, mask=m)
        plsc.store_compressed(wlb_v.at[pl.ds(n, 16)], bv, mask=m)
        return n + plsc.all_reduce_population_count(m)[0]

    n_items = lax.fori_loop(0, BATCH // 16, bgroup, 0)
    ngrp = (n_items + 15) >> 4

    def chunk_i0(c):
        return pl.multiple_of(
            128 * jnp.minimum(lo_blk + CBLK * c, MAXBLK), 128)

    def fire(c, bank):
        pltpu.async_copy(
            table_hbm.at[:, :, pl.ds(chunk_i0(c), CST)],
            bufs_v.at[bank],
            sems.at[bank],
        )

    def extract_hits(h, src_v, base_s, used):
        hsv = hs_v[pl.ds(0, 16)]
        hbv = hb_v[pl.ds(0, 16)]
        for l in range(16):
            cond = l < h

            @pl.when(jnp.logical_and(cond, (used >> l) & 1 > 0))
            def _():
                pltpu.make_async_copy(
                    out_hbm.at[pl.ds(0, N_ACTIONS)],
                    rows_v.at[l],
                    osems.at[l],
                ).wait()

            @pl.when(cond)
            def _():
                il = jnp.full((16,), hsv[l] - base_s, jnp.int32)
                for q in range(NQ):
                    vals = plsc.load_gather(src_v, [phi[q], plo[q], il])
                    rows_v[l, pl.ds(q * 16, 16)] = vals
                pltpu.async_copy(
                    rows_v.at[l],
                    out_hbm.at[pl.ds(hbv[l] * N_ACTIONS, N_ACTIONS)],
                    osems.at[l],
                )

    def scan(c, bank, used):
        i0 = chunk_i0(c)
        pltpu.make_async_copy(
            table_hbm.at[:, :, pl.ds(0, CST)],
            bufs_v.at[bank],
            sems.at[bank],
        ).wait()

        def sgroup(g, used):
            sv = wls_v[pl.ds(g * 16, 16)]
            bv = wlb_v[pl.ds(g * 16, 16)]
            valid = (lanes + g * 16) < n_items
            m = jnp.logical_and(
                jnp.logical_and(sv >= i0, sv < i0 + CST), valid)
            h = plsc.all_reduce_population_count(m)[0]

            @pl.when(h > 0)
            def _():
                plsc.store_compressed(hs_v.at[pl.ds(0, 16)], sv, mask=m)
                plsc.store_compressed(hb_v.at[pl.ds(0, 16)], bv, mask=m)
                extract_hits(h, bufs_v.at[bank], i0, used)

            return used | ((1 << h) - 1)

        return lax.fori_loop(0, ngrp, sgroup, used)

    for k in range(NBANK - 1):
        fire(k, k)

    def cstep(d, used):
        c0 = NBANK * d
        for k in range(NBANK):
            fire(c0 + k + NBANK - 1, (k + NBANK - 1) % NBANK)
            used = scan(c0 + k, k, used)
        return used

    used = lax.fori_loop(0, NCH // NBANK, cstep, 0)
    for k in range(NBANK - 1):
        pltpu.make_async_copy(
            table_hbm.at[:, :, pl.ds(0, CST)],
            bufs_v.at[(NCH + k) % NBANK],
            sems.at[(NCH + k) % NBANK],
        ).wait()

    def tgroup(g, used):
        sv = wls_v[pl.ds(g * 16, 16)]
        bv = wlb_v[pl.ds(g * 16, 16)]
        valid = (lanes + g * 16) < n_items
        m = jnp.logical_and(sv >= TAIL0, valid)
        h = plsc.all_reduce_population_count(m)[0]

        @pl.when(h > 0)
        def _():
            plsc.store_compressed(hs_v.at[pl.ds(0, 16)], sv, mask=m)
            plsc.store_compressed(hb_v.at[pl.ds(0, 16)], bv, mask=m)
            extract_hits(h, tail_v, TAIL0, used)

        return used | ((1 << h) - 1)

    used = lax.fori_loop(0, ngrp, tgroup, used)

    for l in range(16):
        @pl.when((used >> l) & 1 > 0)
        def _():
            pltpu.make_async_copy(
                out_hbm.at[pl.ds(0, N_ACTIONS)],
                rows_v.at[l],
                osems.at[l],
            ).wait()


@jax.jit
def kernel(x, params):
    xf = x.reshape(BATCH)
    tparams = params.T
    table3 = tparams.reshape(8, 8, N_STATES)
    tail3 = tparams[:, TAIL0:].reshape(8, 8, N_STATES - TAIL0)
    run = pl.kernel(
        _gather_body,
        mesh=plsc.VectorSubcoreMesh(core_axis_name="c", subcore_axis_name="s"),
        out_type=jax.ShapeDtypeStruct((BATCH * N_ACTIONS,), jnp.float32),
        scratch_types=[
            pltpu.VMEM((BATCH,), jnp.int32),
            pltpu.VMEM((WLCAP,), jnp.int32),
            pltpu.VMEM((WLCAP,), jnp.int32),
            pltpu.VMEM((8, 8, N_STATES - TAIL0), jnp.float32),
            pltpu.VMEM((NBANK, 8, 8, CST), jnp.float32),
            pltpu.VMEM((16,), jnp.int32),
            pltpu.VMEM((16,), jnp.int32),
            pltpu.VMEM((16, N_ACTIONS), jnp.float32),
            pltpu.SemaphoreType.DMA((NBANK,)),
            pltpu.SemaphoreType.DMA((16,)),
        ],
        compiler_params=pltpu.CompilerParams(needs_layout_passes=False),
    )
    out1 = run(xf, table3, tail3)
    return out1.reshape(BATCH, N_ACTIONS)


# speedup vs baseline: 1.2509x; 1.2509x over previous
import jax
import jax.numpy as jnp
from jax import lax
from jax.experimental import pallas as pl
from jax.experimental.pallas import tpu as pltpu
from jax.experimental.pallas import tpu_sc as plsc

N_STATES = 1000000
N_ACTIONS = 64
BATCH = 16384

NC = 2
NS = 16
NW = NC * NS
NBLK = (N_STATES + 127) // 128
TAIL0 = (N_STATES // 128) * 128
CBLK = 4
CST = CBLK * 128
MAXBLK = (N_STATES - CST) // 128
NBANK = 2
NCH = 63
NQ = N_ACTIONS // 16
WLCAP = BATCH + 16


def _gather_body(x_hbm, table_hbm, tail_hbm, out_hbm,
                 xs_v, wls_v, wlb_v, tail_v, bufs_v, hs_v, hb_v, rows_v,
                 sems, osems):
    wid = lax.axis_index("s") * NC + lax.axis_index("c")
    lo_blk = (NBLK * wid) >> 5
    lo_s = lo_blk * 128
    hi_s = ((NBLK * (wid + 1)) >> 5) * 128
    pltpu.sync_copy(x_hbm, xs_v)
    pltpu.sync_copy(tail_hbm, tail_v)

    lanes = lax.iota(jnp.int32, 16)
    phi = [(lanes + 16 * q) >> 3 for q in range(NQ)]
    plo = [(lanes + 16 * q) & 7 for q in range(NQ)]

    def bgroup(g, n):
        sv = xs_v[pl.ds(g * 16, 16)]
        bv = lanes + g * 16
        m = jnp.logical_and(sv >= lo_s, sv < hi_s)
        plsc.store_compressed(wls_v.at[pl.ds(n, 16)], sv, mask=m)
        plsc.store_compressed(wlb_v.at[pl.ds(n, 16)], bv, mask=m)
        return n + plsc.all_reduce_population_count(m)[0]

    n_items = lax.fori_loop(0, BATCH // 16, bgroup, 0)
    ngrp = (n_items + 15) >> 4

    def chunk_i0(c):
        return pl.multiple_of(
            128 * jnp.minimum(lo_blk + CBLK * c, MAXBLK), 128)

    def fire(c, bank):
        i0 = chunk_i0(c)
        for ct in range(8):
            pltpu.async_copy(
                table_hbm.at[ct, :, pl.ds(i0, CST)],
                bufs_v.at[bank, ct],
                sems.at[bank],
            )

    def extract_hits(h, src_v, base_s, used):
        hsv = hs_v[pl.ds(0, 16)]
        hbv = hb_v[pl.ds(0, 16)]
        for l in range(16):
            cond = l < h

            @pl.when(jnp.logical_and(cond, (used >> l) & 1 > 0))
            def _():
                pltpu.make_async_copy(
                    out_hbm.at[pl.ds(0, N_ACTIONS)],
                    rows_v.at[l],
                    osems.at[l],
                ).wait()

            @pl.when(cond)
            def _():
                il = jnp.full((16,), hsv[l] - base_s, jnp.int32)
                for q in range(NQ):
                    vals = plsc.load_gather(src_v, [phi[q], plo[q], il])
                    rows_v[l, pl.ds(q * 16, 16)] = vals
                pltpu.async_copy(
                    rows_v.at[l],
                    out_hbm.at[pl.ds(hbv[l] * N_ACTIONS, N_ACTIONS)],
                    osems.at[l],
                )

    def scan(c, bank, used):
        i0 = chunk_i0(c)
        pltpu.make_async_copy(
            table_hbm.at[:, :, pl.ds(0, CST)],
            bufs_v.at[bank],
            sems.at[bank],
        ).wait()

        def sgroup(g, used):
            sv = wls_v[pl.ds(g * 16, 16)]
            bv = wlb_v[pl.ds(g * 16, 16)]
            valid = (lanes + g * 16) < n_items
            m = jnp.logical_and(
                jnp.logical_and(sv >= i0, sv < i0 + CST), valid)
            h = plsc.all_reduce_population_count(m)[0]

            @pl.when(h > 0)
            def _():
                plsc.store_compressed(hs_v.at[pl.ds(0, 16)], sv, mask=m)
                plsc.store_compressed(hb_v.at[pl.ds(0, 16)], bv, mask=m)
                extract_hits(h, bufs_v.at[bank], i0, used)

            return used | ((1 << h) - 1)

        return lax.fori_loop(0, ngrp, sgroup, used)

    for k in range(NBANK - 1):
        fire(k, k)

    def cstep(d, used):
        c0 = NBANK * d
        for k in range(NBANK):
            fire(c0 + k + NBANK - 1, (k + NBANK - 1) % NBANK)
            used = scan(c0 + k, k, used)
        return used

    used = lax.fori_loop(0, NCH // NBANK, cstep, 0)
    for k in range(NBANK - 1):
        pltpu.make_async_copy(
            table_hbm.at[:, :, pl.ds(0, CST)],
            bufs_v.at[k],
            sems.at[k],
        ).wait()

    def tgroup(g, used):
        sv = wls_v[pl.ds(g * 16, 16)]
        bv = wlb_v[pl.ds(g * 16, 16)]
        valid = (lanes + g * 16) < n_items
        m = jnp.logical_and(sv >= TAIL0, valid)
        h = plsc.all_reduce_population_count(m)[0]

        @pl.when(h > 0)
        def _():
            plsc.store_compressed(hs_v.at[pl.ds(0, 16)], sv, mask=m)
            plsc.store_compressed(hb_v.at[pl.ds(0, 16)], bv, mask=m)
            extract_hits(h, tail_v, TAIL0, used)

        return used | ((1 << h) - 1)

    used = lax.fori_loop(0, ngrp, tgroup, used)

    for l in range(16):
        @pl.when((used >> l) & 1 > 0)
        def _():
            pltpu.make_async_copy(
                out_hbm.at[pl.ds(0, N_ACTIONS)],
                rows_v.at[l],
                osems.at[l],
            ).wait()


@jax.jit
def kernel(x, params):
    xf = x.reshape(BATCH)
    tparams = params.T
    table3 = tparams.reshape(8, 8, N_STATES)
    tail3 = tparams[:, TAIL0:].reshape(8, 8, N_STATES - TAIL0)
    run = pl.kernel(
        _gather_body,
        mesh=plsc.VectorSubcoreMesh(core_axis_name="c", subcore_axis_name="s"),
        out_type=jax.ShapeDtypeStruct((BATCH * N_ACTIONS,), jnp.float32),
        scratch_types=[
            pltpu.VMEM((BATCH,), jnp.int32),
            pltpu.VMEM((WLCAP,), jnp.int32),
            pltpu.VMEM((WLCAP,), jnp.int32),
            pltpu.VMEM((8, 8, N_STATES - TAIL0), jnp.float32),
            pltpu.VMEM((NBANK, 8, 8, CST), jnp.float32),
            pltpu.VMEM((16,), jnp.int32),
            pltpu.VMEM((16,), jnp.int32),
            pltpu.VMEM((16, N_ACTIONS), jnp.float32),
            pltpu.SemaphoreType.DMA((NBANK,)),
            pltpu.SemaphoreType.DMA((16,)),
        ],
        compiler_params=pltpu.CompilerParams(needs_layout_passes=False),
    )
    out1 = run(xf, table3, tail3)
    return out1.reshape(BATCH, N_ACTIONS)
